# Initial kernel scaffold; baseline (speedup 1.0000x reference)
import jax
import jax.numpy as jnp
from jax.experimental import pallas as pl


def kernel(classification, regression, anchors):
    raise NotImplementedError("write your pallas kernel here")


# speedup vs baseline: 16.8084x; 16.8084x over previous
import functools

import jax
import jax.numpy as jnp
from jax import lax
from jax.experimental import pallas as pl
from jax.experimental.pallas import tpu as pltpu

N_ANCHORS = 5000
N_PAD = 5120
ROWS, COLS = 40, 128
K_MAX = 100
IMG_H = IMG_W = 1024.0
SCORE_THRESH = 0.5
IOU_THRESH = 0.1
NEG = -1.0e30


def _nms_body(ax1, ay1, ax2, ay2, r0, r1, r2, r3, sc, out_ref):
    ax1 = ax1[...]
    ay1 = ay1[...]
    ax2 = ax2[...]
    ay2 = ay2[...]
    widths = ax2 - ax1
    heights = ay2 - ay1
    ctr_x = ax1 + 0.5 * widths
    ctr_y = ay1 + 0.5 * heights
    dx = r0[...] * 0.1
    dy = r1[...] * 0.1
    dw = r2[...] * 0.2
    dh = r3[...] * 0.2
    pred_ctr_x = ctr_x + dx * widths
    pred_ctr_y = ctr_y + dy * heights
    pred_w = jnp.exp(dw) * widths
    pred_h = jnp.exp(dh) * heights
    x1 = jnp.clip(pred_ctr_x - 0.5 * pred_w, 0.0, IMG_W)
    y1 = jnp.clip(pred_ctr_y - 0.5 * pred_h, 0.0, IMG_H)
    x2 = jnp.clip(pred_ctr_x + 0.5 * pred_w, 0.0, IMG_W)
    y2 = jnp.clip(pred_ctr_y + 0.5 * pred_h, 0.0, IMG_H)
    areas = jnp.maximum(x2 - x1, 0.0) * jnp.maximum(y2 - y1, 0.0)

    scores = sc[...]
    masked0 = jnp.where(scores > SCORE_THRESH, scores, NEG)

    idx = (lax.broadcasted_iota(jnp.int32, (ROWS, COLS), 0) * COLS
           + lax.broadcasted_iota(jnp.int32, (ROWS, COLS), 1))
    lane = lax.broadcasted_iota(jnp.int32, (1, COLS), 1)

    out_ref[...] = jnp.zeros_like(out_ref)

    def body(_, state):
        masked, nk = state
        m = jnp.max(masked)
        has = m > NEG * 0.5
        sel = masked == m
        i = jnp.min(jnp.where(sel, idx, jnp.int32(1 << 30)))
        eq = idx == i
        bx1 = jnp.max(jnp.where(eq, x1, NEG))
        by1 = jnp.max(jnp.where(eq, y1, NEG))
        bx2 = jnp.max(jnp.where(eq, x2, NEG))
        by2 = jnp.max(jnp.where(eq, y2, NEG))
        bar = jnp.max(jnp.where(eq, areas, NEG))
        xx1 = jnp.maximum(bx1, x1)
        yy1 = jnp.maximum(by1, y1)
        xx2 = jnp.minimum(bx2, x2)
        yy2 = jnp.minimum(by2, y2)
        inter = jnp.maximum(xx2 - xx1, 0.0) * jnp.maximum(yy2 - yy1, 0.0)
        iou = inter / jnp.maximum(bar + areas - inter, 1e-9)
        masked = jnp.where(has & (iou > IOU_THRESH), NEG, masked)
        row = jnp.where(lane == 0, bx1,
              jnp.where(lane == 1, by1,
              jnp.where(lane == 2, bx2,
              jnp.where(lane == 3, by2,
              jnp.where(lane == 4, m, 0.0)))))
        row = jnp.where(has, row, 0.0)
        out_ref[pl.ds(nk, 1), :] = row
        nk = nk + has.astype(jnp.int32)
        return masked, nk

    lax.fori_loop(0, K_MAX, body, (masked0, jnp.int32(0)))


def _pad2d(v):
    return jnp.pad(v, (0, N_PAD - N_ANCHORS)).reshape(ROWS, COLS)


@jax.jit
def kernel(classification, regression, anchors):
    a = anchors[0]
    r = regression[0]
    parts = [_pad2d(a[:, k]) for k in range(4)]
    parts += [_pad2d(r[:, k]) for k in range(4)]
    parts.append(_pad2d(classification[0, :, 1]))
    out = pl.pallas_call(
        _nms_body,
        out_shape=jax.ShapeDtypeStruct((104, COLS), jnp.float32),
        in_specs=[pl.BlockSpec((ROWS, COLS), lambda: (0, 0))] * 9,
        out_specs=pl.BlockSpec((104, COLS), lambda: (0, 0)),
    )(*parts)
    return out[:K_MAX, :5]
